# baseline (speedup 1.0000x reference)
import functools

import jax
import jax.numpy as jnp
from jax import lax
from jax.experimental import pallas as pl
from jax.experimental.pallas import tpu as pltpu
from jax.experimental.pallas import tpu_sc as plsc

NC, NS = 2, 16
NW = NC * NS


def kernel(x, idxs):
    B, C, T, H, W = x.shape
    xt = jnp.transpose(x, (0, 2, 3, 4, 1))
    idxs32 = idxs.astype(jnp.int32)

    SLABS = B * T
    SPW = SLABS // NW
    HCH = 4
    NCHK = H // HCH
    NFLAT = SPW * NCHK
    NBUF = 3

    mesh = plsc.VectorSubcoreMesh(core_axis_name="c", subcore_axis_name="s")

    @functools.partial(
        pl.kernel,
        mesh=mesh,
        out_type=jax.ShapeDtypeStruct((B, T, H, W, C), jnp.float32),
        scratch_types=[
            pltpu.VMEM((NBUF, HCH, W, C), jnp.float32),
            pltpu.VMEM((T + 16,), jnp.int32),
            pltpu.SemaphoreType.DMA,
            pltpu.SemaphoreType.DMA,
        ],
        compiler_params=pltpu.CompilerParams(use_tc_tiling_on_sc=True),
    )
    def run(x_hbm, idx_hbm, out_hbm, bufs, idx_v, gsem, ssem):
        wid = lax.axis_index("s") * NC + lax.axis_index("c")
        b = wid // (T // SPW)
        tbase = lax.rem(wid, T // SPW) * SPW

        pltpu.sync_copy(idx_hbm, idx_v.at[pl.ds(0, T)])
        tvec = idx_v[pl.ds(tbase, 16)]

        def g_start(c, slot):
            j, h = c // NCHK, c % NCHK
            pltpu.async_copy(
                x_hbm.at[b, tvec[j], pl.ds(h * HCH, HCH)],
                bufs.at[slot],
                gsem,
            )

        def g_wait(slot):
            pltpu.make_async_copy(
                x_hbm.at[b, 0, pl.ds(0, HCH)], bufs.at[slot], gsem
            ).wait()

        def s_start(c, slot):
            j, h = c // NCHK, c % NCHK
            pltpu.async_copy(
                bufs.at[slot],
                out_hbm.at[b, tbase + j, pl.ds(h * HCH, HCH)],
                ssem,
            )

        def s_wait(slot):
            pltpu.make_async_copy(
                bufs.at[slot], out_hbm.at[b, 0, pl.ds(0, HCH)], ssem
            ).wait()

        g_start(0, 0)
        g_start(1, 1)
        for c in range(NFLAT):
            g_wait(c % NBUF)
            s_start(c, c % NBUF)
            if c + 2 < NFLAT:
                if c >= 1:
                    s_wait((c - 1) % NBUF)
                g_start(c + 2, (c + 2) % NBUF)
        for c in range(NFLAT - 3, NFLAT):
            s_wait(c % NBUF)

    out_t = run(xt, idxs32)
    return jnp.transpose(out_t, (0, 4, 1, 2, 3))


# speedup vs baseline: 1.0141x; 1.0141x over previous
import functools

import jax
import jax.numpy as jnp
from jax import lax
from jax.experimental import pallas as pl
from jax.experimental.pallas import tpu as pltpu
from jax.experimental.pallas import tpu_sc as plsc

NC, NS = 2, 16
NW = NC * NS


def kernel(x, idxs):
    B, C, T, H, W = x.shape
    xt = jnp.transpose(x, (0, 2, 3, 4, 1))
    idxs32 = idxs.astype(jnp.int32)

    SLABS = B * T
    SPW = SLABS // NW
    HCH = 8
    NCHK = H // HCH

    mesh = plsc.VectorSubcoreMesh(core_axis_name="c", subcore_axis_name="s")

    @functools.partial(
        pl.kernel,
        mesh=mesh,
        out_type=jax.ShapeDtypeStruct((B, T, H, W, C), jnp.float32),
        scratch_types=[
            pltpu.VMEM((2, HCH, W, C), jnp.float32),
            pltpu.VMEM((T + 16,), jnp.int32),
            pltpu.SemaphoreType.DMA,
        ],
        compiler_params=pltpu.CompilerParams(use_tc_tiling_on_sc=True),
    )
    def run(x_hbm, idx_hbm, out_hbm, bufs, idx_v, sem):
        wid = lax.axis_index("s") * NC + lax.axis_index("c")

        pltpu.sync_copy(idx_hbm, idx_v.at[pl.ds(0, T)])
        tbase = lax.rem(wid, T // SPW) * SPW
        tvec = idx_v[pl.ds(tbase, 16)]

        for j in range(SPW):
            s = wid * SPW + j
            b = s // T
            tout = tbase + j
            tsrc = tvec[j]

            def in_start(c, slot):
                pltpu.async_copy(
                    x_hbm.at[b, tsrc, pl.ds(c * HCH, HCH)],
                    bufs.at[slot],
                    sem,
                )

            def in_wait(slot):
                pltpu.make_async_copy(
                    x_hbm.at[b, 0, pl.ds(0, HCH)],
                    bufs.at[slot],
                    sem,
                ).wait()

            in_start(0, 0)
            for c in range(NCHK):
                if c + 1 < NCHK:
                    in_start(c + 1, (c + 1) % 2)
                in_wait(c % 2)
                pltpu.sync_copy(
                    bufs.at[c % 2],
                    out_hbm.at[b, tout, pl.ds(c * HCH, HCH)],
                )

    out_t = run(xt, idxs32)
    return jnp.transpose(out_t, (0, 4, 1, 2, 3))
